# Initial kernel scaffold; baseline (speedup 1.0000x reference)
import jax
import jax.numpy as jnp
from jax.experimental import pallas as pl


def kernel(x, edge_index, edge_attr, override_candidates, W1a, b1a, W1b, b1b, root1, bias1, W2a, b2a, W2b, b2b, root2, bias2, We1, be1, We2, be2):
    raise NotImplementedError("write your pallas kernel here")


# speedup vs baseline: 1.8059x; 1.8059x over previous
import functools

import jax
import jax.numpy as jnp
from jax import lax
from jax.experimental import pallas as pl
from jax.experimental.pallas import tpu as pltpu
from jax.experimental.pallas import tpu_sc as plsc

_N = 10000
_E = 160000
_P = 100000
_IN = 128
_H = 4

_NC = 2
_NS = 16
_NW = _NC * _NS


def _mesh():
    return plsc.VectorSubcoreMesh(core_axis_name="c", subcore_axis_name="s")


def _sc_gather_rows(table, idx, D, chunk, n_chunks, last_base):
    B = idx.shape[0]

    @functools.partial(
        pl.kernel,
        mesh=_mesh(),
        out_type=jax.ShapeDtypeStruct((B, D), jnp.float32),
        scratch_types=[
            pltpu.VMEM((chunk,), jnp.int32),
            pltpu.VMEM((chunk, D), jnp.float32),
            pltpu.SemaphoreType.DMA,
        ],
    )
    def k(table_ref, idx_ref, out_ref, idx_v, rows_v, sem):
        wid = lax.axis_index("s") * _NC + lax.axis_index("c")
        wbase = wid * (n_chunks * chunk)

        def body(c, carry):
            b = jnp.minimum(wbase + c * chunk, last_base)
            pltpu.sync_copy(idx_ref.at[pl.ds(b, chunk)], idx_v)
            pltpu.async_copy(table_ref.at[idx_v], rows_v, sem).wait()
            pltpu.sync_copy(rows_v, out_ref.at[pl.ds(b, chunk)])
            return carry

        lax.fori_loop(0, n_chunks, body, 0)

    return k(table, idx)


def _sc_gather_elems(table_flat, idx):
    B = idx.shape[0]
    epw = B // _NW

    @functools.partial(
        pl.kernel,
        mesh=_mesh(),
        out_type=jax.ShapeDtypeStruct((B,), jnp.float32),
        scratch_types=[
            pltpu.VMEM((epw,), jnp.int32),
            pltpu.VMEM((epw,), jnp.float32),
            pltpu.SemaphoreType.DMA,
        ],
    )
    def k(table_ref, idx_ref, out_ref, idx_v, vals_v, sem):
        wid = lax.axis_index("s") * _NC + lax.axis_index("c")
        base = wid * epw
        pltpu.sync_copy(idx_ref.at[pl.ds(base, epw)], idx_v)
        pltpu.async_copy(table_ref.at[idx_v], vals_v, sem).wait()
        pltpu.sync_copy(vals_v, out_ref.at[pl.ds(base, epw)])

    return k(table_flat, idx)


def _sc_scatter_add(upd_flat, idx_flat, zeros_flat):
    B = upd_flat.shape[0]
    V = zeros_flat.shape[0]
    epw = B // _NW

    @functools.partial(
        pl.kernel,
        mesh=_mesh(),
        out_type=jax.ShapeDtypeStruct((_NC, V), jnp.float32),
        scratch_types=[
            pltpu.VMEM((epw,), jnp.int32),
            pltpu.VMEM((epw,), jnp.float32),
            pltpu.VMEM_SHARED((V,), jnp.float32),
        ],
    )
    def k(upd_ref, idx_ref, zero_ref, out_ref, idx_v, upd_v, aggr_sh):
        c = lax.axis_index("c")
        s = lax.axis_index("s")
        wid = s * _NC + c
        base = wid * epw

        @pl.when(s == 0)
        def _():
            pltpu.sync_copy(zero_ref, aggr_sh)

        plsc.subcore_barrier()
        pltpu.sync_copy(idx_ref.at[pl.ds(base, epw)], idx_v)
        pltpu.sync_copy(upd_ref.at[pl.ds(base, epw)], upd_v)
        pltpu.sync_copy(upd_v, aggr_sh.at[idx_v], add=True)
        plsc.subcore_barrier()

        @pl.when(s == 0)
        def _():
            pltpu.sync_copy(aggr_sh, out_ref.at[c])

    return k(upd_flat, idx_flat, zeros_flat)


def _tc_edge1(ea, x_src, W1a, b1a, W1bp, b1bp):
    TE = 1000
    grid = _E // TE

    def body(ea_ref, xs_ref, wa_ref, ba_ref, wb_ref, bb_ref, out_ref):
        h = jnp.maximum(
            jnp.dot(ea_ref[...], wa_ref[...], preferred_element_type=jnp.float32)
            + ba_ref[...], 0.0)
        w = jnp.dot(h, wb_ref[...], preferred_element_type=jnp.float32) + bb_ref[...]
        xs = xs_ref[...]
        cols = [
            jnp.sum(xs * w[:, o * _IN:(o + 1) * _IN], axis=1, keepdims=True)
            for o in range(_H)
        ]
        out_ref[...] = jnp.concatenate(cols, axis=1)

    return pl.pallas_call(
        body,
        grid=(grid,),
        in_specs=[
            pl.BlockSpec((TE, 4), lambda i: (i, 0)),
            pl.BlockSpec((TE, _IN), lambda i: (i, 0)),
            pl.BlockSpec((4, 512), lambda i: (0, 0)),
            pl.BlockSpec((1, 512), lambda i: (0, 0)),
            pl.BlockSpec((512, 512), lambda i: (0, 0)),
            pl.BlockSpec((1, 512), lambda i: (0, 0)),
        ],
        out_specs=pl.BlockSpec((TE, _H), lambda i: (i, 0)),
        out_shape=jax.ShapeDtypeStruct((_E, _H), jnp.float32),
    )(ea, x_src, W1a, b1a, W1bp, b1bp)


def _tc_edge2(ea, h1_src, W2a, b2a, W2bp, b2bp):
    TE = 4000
    grid = _E // TE

    def body(ea_ref, hs_ref, wa_ref, ba_ref, wb_ref, bb_ref, out_ref):
        g = jnp.maximum(
            jnp.dot(ea_ref[...], wa_ref[...], preferred_element_type=jnp.float32)
            + ba_ref[...], 0.0)
        w = jnp.dot(g, wb_ref[...], preferred_element_type=jnp.float32) + bb_ref[...]
        hs = hs_ref[...]
        cols = [
            jnp.sum(hs * w[:, o * _H:(o + 1) * _H], axis=1, keepdims=True)
            for o in range(_H)
        ]
        out_ref[...] = jnp.concatenate(cols, axis=1)

    return pl.pallas_call(
        body,
        grid=(grid,),
        in_specs=[
            pl.BlockSpec((TE, 4), lambda i: (i, 0)),
            pl.BlockSpec((TE, _H), lambda i: (i, 0)),
            pl.BlockSpec((4, 16), lambda i: (0, 0)),
            pl.BlockSpec((1, 16), lambda i: (0, 0)),
            pl.BlockSpec((16, 16), lambda i: (0, 0)),
            pl.BlockSpec((1, 16), lambda i: (0, 0)),
        ],
        out_specs=pl.BlockSpec((TE, _H), lambda i: (i, 0)),
        out_shape=jax.ShapeDtypeStruct((_E, _H), jnp.float32),
    )(ea, h1_src, W2a, b2a, W2bp, b2bp)


def _tc_node(partials, xin, root, bias):
    TN = 2000
    grid = _N // TN
    K = xin.shape[1]

    def body(p0_ref, p1_ref, x_ref, r_ref, b_ref, out_ref):
        acc = (p0_ref[0] + p1_ref[0]
               + jnp.dot(x_ref[...], r_ref[...], preferred_element_type=jnp.float32)
               + b_ref[...])
        out_ref[...] = jnp.maximum(acc, 0.0)

    return pl.pallas_call(
        body,
        grid=(grid,),
        in_specs=[
            pl.BlockSpec((1, TN, _H), lambda i: (0, i, 0)),
            pl.BlockSpec((1, TN, _H), lambda i: (1, i, 0)),
            pl.BlockSpec((TN, K), lambda i: (i, 0)),
            pl.BlockSpec((K, _H), lambda i: (0, 0)),
            pl.BlockSpec((1, _H), lambda i: (0, 0)),
        ],
        out_specs=pl.BlockSpec((TN, _H), lambda i: (i, 0)),
        out_shape=jax.ShapeDtypeStruct((_N, _H), jnp.float32),
    )(partials, partials, xin, root, bias)


def _tc_cand(nf, We1, be1, We2, be2):
    TP = 2000
    grid = _P // TP

    def body(nf_ref, w1_ref, b1_ref, w2_ref, b2_ref, out_ref):
        hid = jnp.maximum(
            jnp.dot(nf_ref[...], w1_ref[...], preferred_element_type=jnp.float32)
            + b1_ref[...], 0.0)
        out_ref[...] = (
            jnp.dot(hid, w2_ref[...], preferred_element_type=jnp.float32)
            + b2_ref[...])

    return pl.pallas_call(
        body,
        grid=(grid,),
        in_specs=[
            pl.BlockSpec((TP, 2 * _H), lambda i: (i, 0)),
            pl.BlockSpec((2 * _H, _H), lambda i: (0, 0)),
            pl.BlockSpec((1, _H), lambda i: (0, 0)),
            pl.BlockSpec((_H, _H), lambda i: (0, 0)),
            pl.BlockSpec((1, _H), lambda i: (0, 0)),
        ],
        out_specs=pl.BlockSpec((TP, _H), lambda i: (i, 0)),
        out_shape=jax.ShapeDtypeStruct((_P, _H), jnp.float32),
    )(nf, We1, be1, We2, be2)


def kernel(x, edge_index, edge_attr, override_candidates,
           W1a, b1a, W1b, b1b, root1, bias1,
           W2a, b2a, W2b, b2b, root2, bias2,
           We1, be1, We2, be2):
    src = edge_index[0]
    dst = edge_index[1]

    W1bp = W1b.reshape(512, _IN, _H).transpose(0, 2, 1).reshape(512, 512)
    b1bp = b1b.reshape(_IN, _H).T.reshape(1, 512)
    W2bp = W2b.reshape(16, _H, _H).transpose(0, 2, 1).reshape(16, 16)
    b2bp = b2b.reshape(_H, _H).T.reshape(1, 16)

    zeros_flat = jnp.zeros((_N * _H,), jnp.float32)
    lane4 = jnp.arange(_H, dtype=jnp.int32)
    dst4 = (dst[:, None] * _H + lane4[None, :]).reshape(-1)
    src4 = (src[:, None] * _H + lane4[None, :]).reshape(-1)
    cand4 = (override_candidates.reshape(-1)[:, None] * _H
             + lane4[None, :]).reshape(-1)

    x_src = _sc_gather_rows(x, src, _IN, chunk=1000, n_chunks=5,
                            last_base=_E - 1000)
    msg1 = _tc_edge1(edge_attr, x_src, W1a, b1a.reshape(1, 512), W1bp, b1bp)
    part1 = _sc_scatter_add(msg1.reshape(-1), dst4, zeros_flat)
    h1 = _tc_node(part1.reshape(_NC, _N, _H), x, root1, bias1.reshape(1, _H))

    h1_src = _sc_gather_elems(h1.reshape(-1), src4).reshape(_E, _H)
    msg2 = _tc_edge2(edge_attr, h1_src, W2a, b2a.reshape(1, 16), W2bp, b2bp)
    part2 = _sc_scatter_add(msg2.reshape(-1), dst4, zeros_flat)
    h2 = _tc_node(part2.reshape(_NC, _N, _H), h1, root2, bias2.reshape(1, _H))

    nf = _sc_gather_elems(h2.reshape(-1), cand4).reshape(_P, 2 * _H)
    logits = _tc_cand(nf, We1, be1.reshape(1, _H), We2, be2.reshape(1, _H))

    return (logits, override_candidates, h2)


# speedup vs baseline: 1.8080x; 1.0012x over previous
import functools

import jax
import jax.numpy as jnp
from jax import lax
from jax.experimental import pallas as pl
from jax.experimental.pallas import tpu as pltpu
from jax.experimental.pallas import tpu_sc as plsc

_N = 10000
_E = 160000
_P = 100000
_IN = 128
_H = 4

_NC = 2
_NS = 16
_NW = _NC * _NS


def _mesh():
    return plsc.VectorSubcoreMesh(core_axis_name="c", subcore_axis_name="s")


def _sc_gather_rows(table, idx, D, chunk, n_chunks, last_base):
    B = idx.shape[0]

    @functools.partial(
        pl.kernel,
        mesh=_mesh(),
        out_type=jax.ShapeDtypeStruct((B, D), jnp.float32),
        scratch_types=[
            pltpu.VMEM((chunk,), jnp.int32),
            pltpu.VMEM((chunk, D), jnp.float32),
            pltpu.SemaphoreType.DMA,
        ],
    )
    def k(table_ref, idx_ref, out_ref, idx_v, rows_v, sem):
        wid = lax.axis_index("s") * _NC + lax.axis_index("c")
        wbase = wid * (n_chunks * chunk)

        def body(c, carry):
            b = jnp.minimum(wbase + c * chunk, last_base)
            pltpu.sync_copy(idx_ref.at[pl.ds(b, chunk)], idx_v)
            pltpu.async_copy(table_ref.at[idx_v], rows_v, sem).wait()
            pltpu.sync_copy(rows_v, out_ref.at[pl.ds(b, chunk)])
            return carry

        lax.fori_loop(0, n_chunks, body, 0)

    return k(table, idx)


def _sc_gather_elems(table_flat, idx):
    B = idx.shape[0]
    epw = B // _NW

    @functools.partial(
        pl.kernel,
        mesh=_mesh(),
        out_type=jax.ShapeDtypeStruct((B,), jnp.float32),
        scratch_types=[
            pltpu.VMEM((epw,), jnp.int32),
            pltpu.VMEM((epw,), jnp.float32),
            pltpu.SemaphoreType.DMA,
        ],
    )
    def k(table_ref, idx_ref, out_ref, idx_v, vals_v, sem):
        wid = lax.axis_index("s") * _NC + lax.axis_index("c")
        base = wid * epw
        pltpu.sync_copy(idx_ref.at[pl.ds(base, epw)], idx_v)
        pltpu.async_copy(table_ref.at[idx_v], vals_v, sem).wait()
        pltpu.sync_copy(vals_v, out_ref.at[pl.ds(base, epw)])

    return k(table_flat, idx)


def _sc_scatter_add(upd_flat, idx_flat, zeros_flat):
    B = upd_flat.shape[0]
    V = zeros_flat.shape[0]
    epw = B // _NW

    @functools.partial(
        pl.kernel,
        mesh=_mesh(),
        out_type=jax.ShapeDtypeStruct((_NC, V), jnp.float32),
        scratch_types=[
            pltpu.VMEM((epw,), jnp.int32),
            pltpu.VMEM((epw,), jnp.float32),
            pltpu.VMEM_SHARED((V,), jnp.float32),
        ],
    )
    def k(upd_ref, idx_ref, zero_ref, out_ref, idx_v, upd_v, aggr_sh):
        c = lax.axis_index("c")
        s = lax.axis_index("s")
        wid = s * _NC + c
        base = wid * epw

        @pl.when(s == 0)
        def _():
            pltpu.sync_copy(zero_ref, aggr_sh)

        plsc.subcore_barrier()
        pltpu.sync_copy(idx_ref.at[pl.ds(base, epw)], idx_v)
        pltpu.sync_copy(upd_ref.at[pl.ds(base, epw)], upd_v)
        pltpu.sync_copy(upd_v, aggr_sh.at[idx_v], add=True)
        plsc.subcore_barrier()

        @pl.when(s == 0)
        def _():
            pltpu.sync_copy(aggr_sh, out_ref.at[c])

    return k(upd_flat, idx_flat, zeros_flat)


def _tc_edge1(ea, x_src, W1a, b1a, W1bp, b1bp):
    TE = 1000
    grid = _E // TE

    def body(ea_ref, xs_ref, wa_ref, ba_ref, wb_ref, bb_ref, out_ref):
        h = jnp.maximum(
            jnp.dot(ea_ref[...], wa_ref[...], preferred_element_type=jnp.float32)
            + ba_ref[...], 0.0)
        w = jnp.dot(h.astype(jnp.bfloat16), wb_ref[...],
                    preferred_element_type=jnp.float32) + bb_ref[...]
        xs = xs_ref[...]
        cols = [
            jnp.sum(xs * w[:, o * _IN:(o + 1) * _IN], axis=1, keepdims=True)
            for o in range(_H)
        ]
        out_ref[...] = jnp.concatenate(cols, axis=1)

    return pl.pallas_call(
        body,
        grid=(grid,),
        in_specs=[
            pl.BlockSpec((TE, 4), lambda i: (i, 0)),
            pl.BlockSpec((TE, _IN), lambda i: (i, 0)),
            pl.BlockSpec((4, 512), lambda i: (0, 0)),
            pl.BlockSpec((1, 512), lambda i: (0, 0)),
            pl.BlockSpec((512, 512), lambda i: (0, 0)),
            pl.BlockSpec((1, 512), lambda i: (0, 0)),
        ],
        out_specs=pl.BlockSpec((TE, _H), lambda i: (i, 0)),
        out_shape=jax.ShapeDtypeStruct((_E, _H), jnp.float32),
    )(ea, x_src, W1a, b1a, W1bp, b1bp)


def _tc_edge2(ea, h1_src, W2a, b2a, W2bp, b2bp):
    TE = 4000
    grid = _E // TE

    def body(ea_ref, hs_ref, wa_ref, ba_ref, wb_ref, bb_ref, out_ref):
        g = jnp.maximum(
            jnp.dot(ea_ref[...], wa_ref[...], preferred_element_type=jnp.float32)
            + ba_ref[...], 0.0)
        w = jnp.dot(g, wb_ref[...], preferred_element_type=jnp.float32) + bb_ref[...]
        hs = hs_ref[...]
        cols = [
            jnp.sum(hs * w[:, o * _H:(o + 1) * _H], axis=1, keepdims=True)
            for o in range(_H)
        ]
        out_ref[...] = jnp.concatenate(cols, axis=1)

    return pl.pallas_call(
        body,
        grid=(grid,),
        in_specs=[
            pl.BlockSpec((TE, 4), lambda i: (i, 0)),
            pl.BlockSpec((TE, _H), lambda i: (i, 0)),
            pl.BlockSpec((4, 16), lambda i: (0, 0)),
            pl.BlockSpec((1, 16), lambda i: (0, 0)),
            pl.BlockSpec((16, 16), lambda i: (0, 0)),
            pl.BlockSpec((1, 16), lambda i: (0, 0)),
        ],
        out_specs=pl.BlockSpec((TE, _H), lambda i: (i, 0)),
        out_shape=jax.ShapeDtypeStruct((_E, _H), jnp.float32),
    )(ea, h1_src, W2a, b2a, W2bp, b2bp)


def _tc_node(partials, xin, root, bias):
    TN = 2000
    grid = _N // TN
    K = xin.shape[1]

    def body(p0_ref, p1_ref, x_ref, r_ref, b_ref, out_ref):
        acc = (p0_ref[0] + p1_ref[0]
               + jnp.dot(x_ref[...], r_ref[...], preferred_element_type=jnp.float32)
               + b_ref[...])
        out_ref[...] = jnp.maximum(acc, 0.0)

    return pl.pallas_call(
        body,
        grid=(grid,),
        in_specs=[
            pl.BlockSpec((1, TN, _H), lambda i: (0, i, 0)),
            pl.BlockSpec((1, TN, _H), lambda i: (1, i, 0)),
            pl.BlockSpec((TN, K), lambda i: (i, 0)),
            pl.BlockSpec((K, _H), lambda i: (0, 0)),
            pl.BlockSpec((1, _H), lambda i: (0, 0)),
        ],
        out_specs=pl.BlockSpec((TN, _H), lambda i: (i, 0)),
        out_shape=jax.ShapeDtypeStruct((_N, _H), jnp.float32),
    )(partials, partials, xin, root, bias)


def _tc_cand(nf, We1, be1, We2, be2):
    TP = 2000
    grid = _P // TP

    def body(nf_ref, w1_ref, b1_ref, w2_ref, b2_ref, out_ref):
        hid = jnp.maximum(
            jnp.dot(nf_ref[...], w1_ref[...], preferred_element_type=jnp.float32)
            + b1_ref[...], 0.0)
        out_ref[...] = (
            jnp.dot(hid, w2_ref[...], preferred_element_type=jnp.float32)
            + b2_ref[...])

    return pl.pallas_call(
        body,
        grid=(grid,),
        in_specs=[
            pl.BlockSpec((TP, 2 * _H), lambda i: (i, 0)),
            pl.BlockSpec((2 * _H, _H), lambda i: (0, 0)),
            pl.BlockSpec((1, _H), lambda i: (0, 0)),
            pl.BlockSpec((_H, _H), lambda i: (0, 0)),
            pl.BlockSpec((1, _H), lambda i: (0, 0)),
        ],
        out_specs=pl.BlockSpec((TP, _H), lambda i: (i, 0)),
        out_shape=jax.ShapeDtypeStruct((_P, _H), jnp.float32),
    )(nf, We1, be1, We2, be2)


def kernel(x, edge_index, edge_attr, override_candidates,
           W1a, b1a, W1b, b1b, root1, bias1,
           W2a, b2a, W2b, b2b, root2, bias2,
           We1, be1, We2, be2):
    src = edge_index[0]
    dst = edge_index[1]

    W1bp = W1b.reshape(512, _IN, _H).transpose(0, 2, 1).reshape(512, 512)
    W1bp = W1bp.astype(jnp.bfloat16)
    b1bp = b1b.reshape(_IN, _H).T.reshape(1, 512)
    W2bp = W2b.reshape(16, _H, _H).transpose(0, 2, 1).reshape(16, 16)
    b2bp = b2b.reshape(_H, _H).T.reshape(1, 16)

    zeros_flat = jnp.zeros((_N * _H,), jnp.float32)
    lane4 = jnp.arange(_H, dtype=jnp.int32)
    dst4 = (dst[:, None] * _H + lane4[None, :]).reshape(-1)
    src4 = (src[:, None] * _H + lane4[None, :]).reshape(-1)
    cand4 = (override_candidates.reshape(-1)[:, None] * _H
             + lane4[None, :]).reshape(-1)

    x_src = _sc_gather_rows(x, src, _IN, chunk=1000, n_chunks=5,
                            last_base=_E - 1000)
    msg1 = _tc_edge1(edge_attr, x_src, W1a, b1a.reshape(1, 512), W1bp, b1bp)
    part1 = _sc_scatter_add(msg1.reshape(-1), dst4, zeros_flat)
    h1 = _tc_node(part1.reshape(_NC, _N, _H), x, root1, bias1.reshape(1, _H))

    h1_src = _sc_gather_elems(h1.reshape(-1), src4).reshape(_E, _H)
    msg2 = _tc_edge2(edge_attr, h1_src, W2a, b2a.reshape(1, 16), W2bp, b2bp)
    part2 = _sc_scatter_add(msg2.reshape(-1), dst4, zeros_flat)
    h2 = _tc_node(part2.reshape(_NC, _N, _H), h1, root2, bias2.reshape(1, _H))

    nf = _sc_gather_elems(h2.reshape(-1), cand4).reshape(_P, 2 * _H)
    logits = _tc_cand(nf, We1, be1.reshape(1, _H), We2, be2.reshape(1, _H))

    return (logits, override_candidates, h2)


# speedup vs baseline: 5.6376x; 3.1181x over previous
import functools

import jax
import jax.numpy as jnp
from jax import lax
from jax.experimental import pallas as pl
from jax.experimental.pallas import tpu as pltpu
from jax.experimental.pallas import tpu_sc as plsc

_N = 10000
_E = 160000
_P = 100000
_IN = 128
_H = 4

_NC = 2
_NS = 16
_NW = _NC * _NS


def _mesh():
    return plsc.VectorSubcoreMesh(core_axis_name="c", subcore_axis_name="s")


def _sc_gather_rows(table, idx, D, chunk, n_chunks):
    B = idx.shape[0]

    @functools.partial(
        pl.kernel,
        mesh=_mesh(),
        out_type=jax.ShapeDtypeStruct((B, D), jnp.float32),
        scratch_types=[
            pltpu.VMEM((chunk,), jnp.int32),
            pltpu.VMEM((chunk, D), jnp.float32),
            pltpu.SemaphoreType.DMA,
        ],
    )
    def k(table_ref, idx_ref, out_ref, idx_v, rows_v, sem):
        wid = lax.axis_index("s") * _NC + lax.axis_index("c")
        wbase = wid * (n_chunks * chunk)

        def body(c, carry):
            b = wbase + c * chunk
            pltpu.sync_copy(idx_ref.at[pl.ds(b, chunk)], idx_v)
            pltpu.async_copy(table_ref.at[idx_v], rows_v, sem).wait()
            pltpu.sync_copy(rows_v, out_ref.at[pl.ds(b, chunk)])
            return carry

        lax.fori_loop(0, n_chunks, body, 0)

    return k(table, idx)


def _offset_loop(idx_v, seg, off):
    def body(j, carry):
        sl = pl.ds(j * 16, 16)
        idx_v[sl] = idx_v[sl] + off
        return carry
    lax.fori_loop(0, seg // 16, body, 0)
    rem = seg % 16
    if rem:
        sl = pl.ds(seg - 16, 16)
        lane = lax.iota(jnp.int32, 16)
        idx_v[sl] = jnp.where(lane >= 16 - rem, idx_v[sl] + off, idx_v[sl])


def _sc_scatter_add(updT_flat, dst, zeros_flat):
    SEG = _E // 8

    @functools.partial(
        pl.kernel,
        mesh=_mesh(),
        out_type=jax.ShapeDtypeStruct((_NC, _H * _N), jnp.float32),
        scratch_types=[
            pltpu.VMEM((SEG,), jnp.int32),
            pltpu.VMEM((SEG,), jnp.float32),
            pltpu.VMEM_SHARED((_H * _N,), jnp.float32),
        ],
    )
    def k(upd_ref, dst_ref, zero_ref, out_ref, idx_v, upd_v, aggr_sh):
        c = lax.axis_index("c")
        s = lax.axis_index("s")
        wid = s * _NC + c
        o = wid // 8
        p = wid % 8

        @pl.when(s == 0)
        def _():
            pltpu.sync_copy(zero_ref, aggr_sh)

        pltpu.sync_copy(dst_ref.at[pl.ds(p * SEG, SEG)], idx_v)
        _offset_loop(idx_v, SEG, o * _N)
        pltpu.sync_copy(upd_ref.at[pl.ds(o * _E + p * SEG, SEG)], upd_v)
        plsc.subcore_barrier()
        pltpu.sync_copy(upd_v, aggr_sh.at[idx_v], add=True)
        plsc.subcore_barrier()

        @pl.when(s == 0)
        def _():
            pltpu.sync_copy(aggr_sh, out_ref.at[c])

    return k(updT_flat, dst, zeros_flat)


def _sc_gather_nodeT(tableT_flat, idx):
    SEG = _E // 8

    @functools.partial(
        pl.kernel,
        mesh=_mesh(),
        out_type=jax.ShapeDtypeStruct((_H * _E,), jnp.float32),
        scratch_types=[
            pltpu.VMEM((SEG,), jnp.int32),
            pltpu.VMEM((SEG,), jnp.float32),
            pltpu.SemaphoreType.DMA,
        ],
    )
    def k(table_ref, idx_ref, out_ref, idx_v, vals_v, sem):
        wid = lax.axis_index("s") * _NC + lax.axis_index("c")
        o = wid // 8
        p = wid % 8
        pltpu.sync_copy(idx_ref.at[pl.ds(p * SEG, SEG)], idx_v)
        _offset_loop(idx_v, SEG, o * _N)
        pltpu.async_copy(table_ref.at[idx_v], vals_v, sem).wait()
        pltpu.sync_copy(vals_v, out_ref.at[pl.ds(o * _E + p * SEG, SEG)])

    return k(tableT_flat, idx)


def _sc_gather_pairsT(tableT_flat, candT_flat):
    SEG = _P // 4

    @functools.partial(
        pl.kernel,
        mesh=_mesh(),
        out_type=jax.ShapeDtypeStruct((2 * _H * _P,), jnp.float32),
        scratch_types=[
            pltpu.VMEM((SEG,), jnp.int32),
            pltpu.VMEM((SEG,), jnp.float32),
            pltpu.SemaphoreType.DMA,
        ],
    )
    def k(table_ref, cand_ref, out_ref, idx_v, vals_v, sem):
        wid = lax.axis_index("s") * _NC + lax.axis_index("c")
        f = wid // 4
        part = wid % 4
        s = f // _H
        o = f % _H
        pltpu.sync_copy(cand_ref.at[pl.ds(s * _P + part * SEG, SEG)], idx_v)
        _offset_loop(idx_v, SEG, o * _N)
        pltpu.async_copy(table_ref.at[idx_v], vals_v, sem).wait()
        pltpu.sync_copy(vals_v, out_ref.at[pl.ds(f * _P + part * SEG, SEG)])

    return k(tableT_flat, candT_flat)


def _tc_edge1(eaT, x_src, W1a, b1a, W1bp, b1bp):
    TE = 1280
    grid = _E // TE
    dn_t = (((0,), (0,)), ((), ()))

    def body(ea_ref, xs_ref, wa_ref, ba_ref, wb_ref, bb_ref, out_ref):
        h = jnp.maximum(
            lax.dot_general(ea_ref[...], wa_ref[...], dn_t,
                            preferred_element_type=jnp.float32)
            + ba_ref[...], 0.0)
        w = jnp.dot(h, wb_ref[...], preferred_element_type=jnp.float32) \
            + bb_ref[...]
        xs = xs_ref[...]
        cols = [
            jnp.sum(xs * w[:, o * _IN:(o + 1) * _IN], axis=1, keepdims=True)
            for o in range(_H)
        ]
        msg = jnp.concatenate(cols, axis=1)
        eye = jnp.eye(_H, dtype=jnp.float32)
        out_ref[...] = lax.dot_general(
            eye, msg, (((1,), (1,)), ((), ())),
            preferred_element_type=jnp.float32)

    return pl.pallas_call(
        body,
        grid=(grid,),
        in_specs=[
            pl.BlockSpec((_H, TE), lambda i: (0, i)),
            pl.BlockSpec((TE, _IN), lambda i: (i, 0)),
            pl.BlockSpec((4, 512), lambda i: (0, 0)),
            pl.BlockSpec((1, 512), lambda i: (0, 0)),
            pl.BlockSpec((512, 512), lambda i: (0, 0)),
            pl.BlockSpec((1, 512), lambda i: (0, 0)),
        ],
        out_specs=pl.BlockSpec((_H, TE), lambda i: (0, i)),
        out_shape=jax.ShapeDtypeStruct((_H, _E), jnp.float32),
    )(eaT, x_src, W1a, b1a, W1bp, b1bp)


def _tc_edge2(eaT, h1sT, W2aT, b2aT, W2bpT, b2bpT):
    TE = 16000
    grid = _E // TE

    def body(ea_ref, hs_ref, wa_ref, ba_ref, wb_ref, bb_ref, out_ref):
        g = jnp.maximum(
            jnp.dot(wa_ref[...], ea_ref[...],
                    preferred_element_type=jnp.float32) + ba_ref[...], 0.0)
        w = jnp.dot(wb_ref[...], g,
                    preferred_element_type=jnp.float32) + bb_ref[...]
        hs = hs_ref[...]
        rows = []
        for o in range(_H):
            acc = hs[0:1, :] * w[o * _H:o * _H + 1, :]
            for i in range(1, _H):
                acc = acc + hs[i:i + 1, :] * w[o * _H + i:o * _H + i + 1, :]
            rows.append(acc)
        out_ref[...] = jnp.concatenate(rows, axis=0)

    return pl.pallas_call(
        body,
        grid=(grid,),
        in_specs=[
            pl.BlockSpec((_H, TE), lambda i: (0, i)),
            pl.BlockSpec((_H, TE), lambda i: (0, i)),
            pl.BlockSpec((16, 4), lambda i: (0, 0)),
            pl.BlockSpec((16, 1), lambda i: (0, 0)),
            pl.BlockSpec((16, 16), lambda i: (0, 0)),
            pl.BlockSpec((16, 1), lambda i: (0, 0)),
        ],
        out_specs=pl.BlockSpec((_H, TE), lambda i: (0, i)),
        out_shape=jax.ShapeDtypeStruct((_H, _E), jnp.float32),
    )(eaT, h1sT, W2aT, b2aT, W2bpT, b2bpT)


def _tc_node(partials, xinT, rootT, biasc):
    TN = _N
    grid = 1
    K = xinT.shape[0]

    def body(p0_ref, p1_ref, x_ref, r_ref, b_ref, out_ref):
        acc = (p0_ref[0] + p1_ref[0]
               + jnp.dot(r_ref[...], x_ref[...],
                         preferred_element_type=jnp.float32)
               + b_ref[...])
        out_ref[...] = jnp.maximum(acc, 0.0)

    return pl.pallas_call(
        body,
        grid=(grid,),
        in_specs=[
            pl.BlockSpec((1, _H, TN), lambda i: (0, 0, i)),
            pl.BlockSpec((1, _H, TN), lambda i: (1, 0, i)),
            pl.BlockSpec((K, TN), lambda i: (0, i)),
            pl.BlockSpec((_H, K), lambda i: (0, 0)),
            pl.BlockSpec((_H, 1), lambda i: (0, 0)),
        ],
        out_specs=pl.BlockSpec((_H, TN), lambda i: (0, i)),
        out_shape=jax.ShapeDtypeStruct((_H, _N), jnp.float32),
    )(partials, partials, xinT, rootT, biasc)


def _tc_cand(nfT, We1T, be1c, We2T, be2c):
    TP = _P
    grid = 1

    def body(nf_ref, w1_ref, b1_ref, w2_ref, b2_ref, out_ref):
        hid = jnp.maximum(
            jnp.dot(w1_ref[...], nf_ref[...],
                    preferred_element_type=jnp.float32) + b1_ref[...], 0.0)
        out_ref[...] = jnp.dot(w2_ref[...], hid,
                               preferred_element_type=jnp.float32) + b2_ref[...]

    return pl.pallas_call(
        body,
        grid=(grid,),
        in_specs=[
            pl.BlockSpec((2 * _H, TP), lambda i: (0, i)),
            pl.BlockSpec((_H, 2 * _H), lambda i: (0, 0)),
            pl.BlockSpec((_H, 1), lambda i: (0, 0)),
            pl.BlockSpec((_H, _H), lambda i: (0, 0)),
            pl.BlockSpec((_H, 1), lambda i: (0, 0)),
        ],
        out_specs=pl.BlockSpec((_H, TP), lambda i: (0, i)),
        out_shape=jax.ShapeDtypeStruct((_H, _P), jnp.float32),
    )(nfT, We1T, be1c, We2T, be2c)


def kernel(x, edge_index, edge_attr, override_candidates,
           W1a, b1a, W1b, b1b, root1, bias1,
           W2a, b2a, W2b, b2b, root2, bias2,
           We1, be1, We2, be2):
    src = edge_index[0]
    dst = edge_index[1]

    eaT = edge_attr.T
    xT = x.T
    candT_flat = override_candidates.T.reshape(-1)

    W1bp = W1b.reshape(512, _IN, _H).transpose(0, 2, 1).reshape(512, 512)
    b1bp = b1b.reshape(_IN, _H).T.reshape(1, 512)
    W2aT = W2a.T
    b2aT = b2a.reshape(16, 1)
    W2bpT = W2b.reshape(16, _H, _H).transpose(2, 1, 0).reshape(16, 16)
    b2bpT = b2b.reshape(_H, _H).T.reshape(16, 1)

    zeros_flat = jnp.zeros((_H * _N,), jnp.float32)

    x_src = _sc_gather_rows(x, src, _IN, chunk=1000, n_chunks=5)
    msg1T = _tc_edge1(eaT, x_src, W1a, b1a.reshape(1, 512), W1bp, b1bp)
    part1 = _sc_scatter_add(msg1T.reshape(-1), dst, zeros_flat)
    h1T = _tc_node(part1.reshape(_NC, _H, _N), xT, root1.T,
                   bias1.reshape(_H, 1))

    h1sT = _sc_gather_nodeT(h1T.reshape(-1), src)
    msg2T = _tc_edge2(eaT, h1sT.reshape(_H, _E), W2aT, b2aT, W2bpT, b2bpT)
    part2 = _sc_scatter_add(msg2T.reshape(-1), dst, zeros_flat)
    h2T = _tc_node(part2.reshape(_NC, _H, _N), h1T, root2.T,
                   bias2.reshape(_H, 1))

    nfT = _sc_gather_pairsT(h2T.reshape(-1), candT_flat)
    logitsT = _tc_cand(nfT.reshape(2 * _H, _P), We1.T,
                       be1.reshape(_H, 1), We2.T, be2.reshape(_H, 1))

    return (logitsT.T, override_candidates, h2T.T)


# speedup vs baseline: 6.4880x; 1.1508x over previous
import functools

import jax
import jax.numpy as jnp
from jax import lax
from jax.experimental import pallas as pl
from jax.experimental.pallas import tpu as pltpu
from jax.experimental.pallas import tpu_sc as plsc

_N = 10000
_E = 160000
_P = 100000
_IN = 128
_H = 4

_NC = 2
_NS = 16
_NW = _NC * _NS


def _mesh():
    return plsc.VectorSubcoreMesh(core_axis_name="c", subcore_axis_name="s")


def _sc_gather_rows(table, idx, D, chunk, n_chunks):
    B = idx.shape[0]

    @functools.partial(
        pl.kernel,
        mesh=_mesh(),
        out_type=jax.ShapeDtypeStruct((B, D), jnp.float32),
        scratch_types=[
            pltpu.VMEM((chunk,), jnp.int32),
            pltpu.VMEM((chunk, D), jnp.float32),
            pltpu.SemaphoreType.DMA,
        ],
    )
    def k(table_ref, idx_ref, out_ref, idx_v, rows_v, sem):
        wid = lax.axis_index("s") * _NC + lax.axis_index("c")
        wbase = wid * (n_chunks * chunk)

        def body(c, carry):
            b = wbase + c * chunk
            pltpu.sync_copy(idx_ref.at[pl.ds(b, chunk)], idx_v)
            pltpu.async_copy(table_ref.at[idx_v], rows_v, sem).wait()
            pltpu.sync_copy(rows_v, out_ref.at[pl.ds(b, chunk)])
            return carry

        lax.fori_loop(0, n_chunks, body, 0)

    return k(table, idx)


def _offset_loop(idx_v, seg, off):
    def body(j, carry):
        sl = pl.ds(j * 16, 16)
        idx_v[sl] = idx_v[sl] + off
        return carry
    lax.fori_loop(0, seg // 16, body, 0)
    rem = seg % 16
    if rem:
        sl = pl.ds(seg - 16, 16)
        lane = lax.iota(jnp.int32, 16)
        idx_v[sl] = jnp.where(lane >= 16 - rem, idx_v[sl] + off, idx_v[sl])


def _sc_scatter_add(updT_flat, dst, zeros_flat):
    SEG = _E // 8

    @functools.partial(
        pl.kernel,
        mesh=_mesh(),
        out_type=jax.ShapeDtypeStruct((_NC, _H * _N), jnp.float32),
        scratch_types=[
            pltpu.VMEM((SEG,), jnp.int32),
            pltpu.VMEM((SEG,), jnp.float32),
            pltpu.VMEM_SHARED((_H * _N,), jnp.float32),
        ],
    )
    def k(upd_ref, dst_ref, zero_ref, out_ref, idx_v, upd_v, aggr_sh):
        c = lax.axis_index("c")
        s = lax.axis_index("s")
        wid = s * _NC + c
        o = wid // 8
        p = wid % 8

        @pl.when(s == 0)
        def _():
            pltpu.sync_copy(zero_ref, aggr_sh)

        pltpu.sync_copy(dst_ref.at[pl.ds(p * SEG, SEG)], idx_v)
        _offset_loop(idx_v, SEG, o * _N)
        pltpu.sync_copy(upd_ref.at[pl.ds(o * _E + p * SEG, SEG)], upd_v)
        plsc.subcore_barrier()
        pltpu.sync_copy(upd_v, aggr_sh.at[idx_v], add=True)
        plsc.subcore_barrier()

        @pl.when(s == 0)
        def _():
            pltpu.sync_copy(aggr_sh, out_ref.at[c])

    return k(updT_flat, dst, zeros_flat)


def _sc_gather_nodeT(tableT_flat, idx):
    SEG = _E // 8

    @functools.partial(
        pl.kernel,
        mesh=_mesh(),
        out_type=jax.ShapeDtypeStruct((_H * _E,), jnp.float32),
        scratch_types=[
            pltpu.VMEM((SEG,), jnp.int32),
            pltpu.VMEM((SEG,), jnp.float32),
            pltpu.VMEM_SHARED((_H * _N,), jnp.float32),
            pltpu.SemaphoreType.DMA,
        ],
    )
    def k(table_ref, idx_ref, out_ref, idx_v, vals_v, table_sh, sem):
        wid = lax.axis_index("s") * _NC + lax.axis_index("c")
        s = lax.axis_index("s")
        o = wid // 8
        p = wid % 8

        @pl.when(s == 0)
        def _():
            pltpu.sync_copy(table_ref, table_sh)

        pltpu.sync_copy(idx_ref.at[pl.ds(p * SEG, SEG)], idx_v)
        _offset_loop(idx_v, SEG, o * _N)
        plsc.subcore_barrier()
        pltpu.async_copy(table_sh.at[idx_v], vals_v, sem).wait()
        pltpu.sync_copy(vals_v, out_ref.at[pl.ds(o * _E + p * SEG, SEG)])

    return k(tableT_flat, idx)


def _sc_gather_pairsT(tableT_flat, candT_flat):
    SEG = _P // 4

    @functools.partial(
        pl.kernel,
        mesh=_mesh(),
        out_type=jax.ShapeDtypeStruct((2 * _H * _P,), jnp.float32),
        scratch_types=[
            pltpu.VMEM((SEG,), jnp.int32),
            pltpu.VMEM((SEG,), jnp.float32),
            pltpu.VMEM_SHARED((_H * _N,), jnp.float32),
            pltpu.SemaphoreType.DMA,
        ],
    )
    def k(table_ref, cand_ref, out_ref, idx_v, vals_v, table_sh, sem):
        wid = lax.axis_index("s") * _NC + lax.axis_index("c")
        sid = lax.axis_index("s")
        f = wid // 4
        part = wid % 4
        s = f // _H
        o = f % _H

        @pl.when(sid == 0)
        def _():
            pltpu.sync_copy(table_ref, table_sh)

        pltpu.sync_copy(cand_ref.at[pl.ds(s * _P + part * SEG, SEG)], idx_v)
        _offset_loop(idx_v, SEG, o * _N)
        plsc.subcore_barrier()
        pltpu.async_copy(table_sh.at[idx_v], vals_v, sem).wait()
        pltpu.sync_copy(vals_v, out_ref.at[pl.ds(f * _P + part * SEG, SEG)])

    return k(tableT_flat, candT_flat)


def _tc_edge1(eaT, x_src, W1a, b1a, W1bp, b1bp):
    TE = 1280
    grid = _E // TE
    dn_t = (((0,), (0,)), ((), ()))

    def body(ea_ref, xs_ref, wa_ref, ba_ref, wb_ref, bb_ref, out_ref):
        h = jnp.maximum(
            lax.dot_general(ea_ref[...], wa_ref[...], dn_t,
                            preferred_element_type=jnp.float32)
            + ba_ref[...], 0.0)
        w = jnp.dot(h.astype(jnp.bfloat16), wb_ref[...],
                    preferred_element_type=jnp.float32) \
            + bb_ref[...]
        xs = xs_ref[...]
        cols = [
            jnp.sum(xs * w[:, o * _IN:(o + 1) * _IN], axis=1, keepdims=True)
            for o in range(_H)
        ]
        msg = jnp.concatenate(cols, axis=1)
        eye = jnp.eye(_H, dtype=jnp.float32)
        out_ref[...] = lax.dot_general(
            eye, msg, (((1,), (1,)), ((), ())),
            preferred_element_type=jnp.float32)

    return pl.pallas_call(
        body,
        grid=(grid,),
        in_specs=[
            pl.BlockSpec((_H, TE), lambda i: (0, i)),
            pl.BlockSpec((TE, _IN), lambda i: (i, 0)),
            pl.BlockSpec((4, 512), lambda i: (0, 0)),
            pl.BlockSpec((1, 512), lambda i: (0, 0)),
            pl.BlockSpec((512, 512), lambda i: (0, 0)),
            pl.BlockSpec((1, 512), lambda i: (0, 0)),
        ],
        out_specs=pl.BlockSpec((_H, TE), lambda i: (0, i)),
        out_shape=jax.ShapeDtypeStruct((_H, _E), jnp.float32),
    )(eaT, x_src, W1a, b1a, W1bp, b1bp)


def _tc_edge2(eaT, h1sT, W2aT, b2aT, W2bpT, b2bpT):
    TE = 16000
    grid = _E // TE

    def body(ea_ref, hs_ref, wa_ref, ba_ref, wb_ref, bb_ref, out_ref):
        g = jnp.maximum(
            jnp.dot(wa_ref[...], ea_ref[...],
                    preferred_element_type=jnp.float32) + ba_ref[...], 0.0)
        w = jnp.dot(wb_ref[...], g,
                    preferred_element_type=jnp.float32) + bb_ref[...]
        hs = hs_ref[...]
        rows = []
        for o in range(_H):
            acc = hs[0:1, :] * w[o * _H:o * _H + 1, :]
            for i in range(1, _H):
                acc = acc + hs[i:i + 1, :] * w[o * _H + i:o * _H + i + 1, :]
            rows.append(acc)
        out_ref[...] = jnp.concatenate(rows, axis=0)

    return pl.pallas_call(
        body,
        grid=(grid,),
        in_specs=[
            pl.BlockSpec((_H, TE), lambda i: (0, i)),
            pl.BlockSpec((_H, TE), lambda i: (0, i)),
            pl.BlockSpec((16, 4), lambda i: (0, 0)),
            pl.BlockSpec((16, 1), lambda i: (0, 0)),
            pl.BlockSpec((16, 16), lambda i: (0, 0)),
            pl.BlockSpec((16, 1), lambda i: (0, 0)),
        ],
        out_specs=pl.BlockSpec((_H, TE), lambda i: (0, i)),
        out_shape=jax.ShapeDtypeStruct((_H, _E), jnp.float32),
    )(eaT, h1sT, W2aT, b2aT, W2bpT, b2bpT)


def _tc_node(partials, xinT, rootT, biasc):
    TN = _N
    grid = 1
    K = xinT.shape[0]

    def body(p0_ref, p1_ref, x_ref, r_ref, b_ref, out_ref):
        acc = (p0_ref[0] + p1_ref[0]
               + jnp.dot(r_ref[...], x_ref[...],
                         preferred_element_type=jnp.float32)
               + b_ref[...])
        out_ref[...] = jnp.maximum(acc, 0.0)

    return pl.pallas_call(
        body,
        grid=(grid,),
        in_specs=[
            pl.BlockSpec((1, _H, TN), lambda i: (0, 0, i)),
            pl.BlockSpec((1, _H, TN), lambda i: (1, 0, i)),
            pl.BlockSpec((K, TN), lambda i: (0, i)),
            pl.BlockSpec((_H, K), lambda i: (0, 0)),
            pl.BlockSpec((_H, 1), lambda i: (0, 0)),
        ],
        out_specs=pl.BlockSpec((_H, TN), lambda i: (0, i)),
        out_shape=jax.ShapeDtypeStruct((_H, _N), jnp.float32),
    )(partials, partials, xinT, rootT, biasc)


def _tc_cand(nfT, We1T, be1c, We2T, be2c):
    TP = _P
    grid = 1

    def body(nf_ref, w1_ref, b1_ref, w2_ref, b2_ref, out_ref):
        hid = jnp.maximum(
            jnp.dot(w1_ref[...], nf_ref[...],
                    preferred_element_type=jnp.float32) + b1_ref[...], 0.0)
        out_ref[...] = jnp.dot(w2_ref[...], hid,
                               preferred_element_type=jnp.float32) + b2_ref[...]

    return pl.pallas_call(
        body,
        grid=(grid,),
        in_specs=[
            pl.BlockSpec((2 * _H, TP), lambda i: (0, i)),
            pl.BlockSpec((_H, 2 * _H), lambda i: (0, 0)),
            pl.BlockSpec((_H, 1), lambda i: (0, 0)),
            pl.BlockSpec((_H, _H), lambda i: (0, 0)),
            pl.BlockSpec((_H, 1), lambda i: (0, 0)),
        ],
        out_specs=pl.BlockSpec((_H, TP), lambda i: (0, i)),
        out_shape=jax.ShapeDtypeStruct((_H, _P), jnp.float32),
    )(nfT, We1T, be1c, We2T, be2c)


def kernel(x, edge_index, edge_attr, override_candidates,
           W1a, b1a, W1b, b1b, root1, bias1,
           W2a, b2a, W2b, b2b, root2, bias2,
           We1, be1, We2, be2):
    src = edge_index[0]
    dst = edge_index[1]

    eaT = edge_attr.T
    xT = x.T
    candT_flat = override_candidates.T.reshape(-1)

    W1bp = W1b.reshape(512, _IN, _H).transpose(0, 2, 1).reshape(512, 512)
    W1bp = W1bp.astype(jnp.bfloat16)
    b1bp = b1b.reshape(_IN, _H).T.reshape(1, 512)
    W2aT = W2a.T
    b2aT = b2a.reshape(16, 1)
    W2bpT = W2b.reshape(16, _H, _H).transpose(2, 1, 0).reshape(16, 16)
    b2bpT = b2b.reshape(_H, _H).T.reshape(16, 1)

    zeros_flat = jnp.zeros((_H * _N,), jnp.float32)

    x_src = _sc_gather_rows(x, src, _IN, chunk=1000, n_chunks=5)
    msg1T = _tc_edge1(eaT, x_src, W1a, b1a.reshape(1, 512), W1bp, b1bp)
    part1 = _sc_scatter_add(msg1T.reshape(-1), dst, zeros_flat)
    h1T = _tc_node(part1.reshape(_NC, _H, _N), xT, root1.T,
                   bias1.reshape(_H, 1))

    h1sT = _sc_gather_nodeT(h1T.reshape(-1), src)
    msg2T = _tc_edge2(eaT, h1sT.reshape(_H, _E), W2aT, b2aT, W2bpT, b2bpT)
    part2 = _sc_scatter_add(msg2T.reshape(-1), dst, zeros_flat)
    h2T = _tc_node(part2.reshape(_NC, _H, _N), h1T, root2.T,
                   bias2.reshape(_H, 1))

    nfT = _sc_gather_pairsT(h2T.reshape(-1), candT_flat)
    logitsT = _tc_cand(nfT.reshape(2 * _H, _P), We1.T,
                       be1.reshape(_H, 1), We2.T, be2.reshape(_H, 1))

    return (logitsT.T, override_candidates, h2T.T)


# speedup vs baseline: 7.0548x; 1.0874x over previous
import functools

import jax
import jax.numpy as jnp
from jax import lax
from jax.experimental import pallas as pl
from jax.experimental.pallas import tpu as pltpu
from jax.experimental.pallas import tpu_sc as plsc

_N = 10000
_E = 160000
_P = 100000
_IN = 128
_H = 4

_NC = 2
_NS = 16
_NW = _NC * _NS


def _mesh():
    return plsc.VectorSubcoreMesh(core_axis_name="c", subcore_axis_name="s")


def _sc_gather_rows(table, idx, D, chunk, n_chunks):
    B = idx.shape[0]

    @functools.partial(
        pl.kernel,
        mesh=_mesh(),
        out_type=jax.ShapeDtypeStruct((B, D), jnp.float32),
        scratch_types=[
            pltpu.VMEM((chunk,), jnp.int32),
            pltpu.VMEM((chunk, D), jnp.float32),
            pltpu.SemaphoreType.DMA,
        ],
    )
    def k(table_ref, idx_ref, out_ref, idx_v, rows_v, sem):
        wid = lax.axis_index("s") * _NC + lax.axis_index("c")
        wbase = wid * (n_chunks * chunk)

        def body(c, carry):
            b = wbase + c * chunk
            pltpu.sync_copy(idx_ref.at[pl.ds(b, chunk)], idx_v)
            pltpu.async_copy(table_ref.at[idx_v], rows_v, sem).wait()
            pltpu.sync_copy(rows_v, out_ref.at[pl.ds(b, chunk)])
            return carry

        lax.fori_loop(0, n_chunks, body, 0)

    return k(table, idx)


def _offset_loop(idx_v, seg, off):
    def body(j, carry):
        sl = pl.ds(j * 16, 16)
        idx_v[sl] = idx_v[sl] + off
        return carry
    lax.fori_loop(0, seg // 16, body, 0)
    rem = seg % 16
    if rem:
        sl = pl.ds(seg - 16, 16)
        lane = lax.iota(jnp.int32, 16)
        idx_v[sl] = jnp.where(lane >= 16 - rem, idx_v[sl] + off, idx_v[sl])


def _sc_scatter_add(updT_flat, dst, zeros_flat):
    SEG = _E // 8

    @functools.partial(
        pl.kernel,
        mesh=_mesh(),
        out_type=jax.ShapeDtypeStruct((_NC, _H * _N), jnp.float32),
        scratch_types=[
            pltpu.VMEM((SEG,), jnp.int32),
            pltpu.VMEM((SEG,), jnp.float32),
            pltpu.VMEM_SHARED((_H * _N,), jnp.float32),
        ],
    )
    def k(upd_ref, dst_ref, zero_ref, out_ref, idx_v, upd_v, aggr_sh):
        c = lax.axis_index("c")
        s = lax.axis_index("s")
        wid = s * _NC + c
        o = wid // 8
        p = wid % 8

        @pl.when(s == 0)
        def _():
            pltpu.sync_copy(zero_ref, aggr_sh)

        pltpu.sync_copy(dst_ref.at[pl.ds(p * SEG, SEG)], idx_v)
        _offset_loop(idx_v, SEG, o * _N)
        pltpu.sync_copy(upd_ref.at[pl.ds(o * _E + p * SEG, SEG)], upd_v)
        plsc.subcore_barrier()
        pltpu.sync_copy(upd_v, aggr_sh.at[idx_v], add=True)
        plsc.subcore_barrier()

        @pl.when(s == 0)
        def _():
            pltpu.sync_copy(aggr_sh, out_ref.at[c])

    return k(updT_flat, dst, zeros_flat)


def _sc_gather_nodeT(tableT_flat, idx):
    SEG = _E // 8

    @functools.partial(
        pl.kernel,
        mesh=_mesh(),
        out_type=jax.ShapeDtypeStruct((_H * _E,), jnp.float32),
        scratch_types=[
            pltpu.VMEM((SEG,), jnp.int32),
            pltpu.VMEM((SEG,), jnp.float32),
            pltpu.VMEM_SHARED((_H * _N,), jnp.float32),
            pltpu.SemaphoreType.DMA,
        ],
    )
    def k(table_ref, idx_ref, out_ref, idx_v, vals_v, table_sh, sem):
        wid = lax.axis_index("s") * _NC + lax.axis_index("c")
        s = lax.axis_index("s")
        o = wid // 8
        p = wid % 8

        @pl.when(s == 0)
        def _():
            pltpu.sync_copy(table_ref, table_sh)

        pltpu.sync_copy(idx_ref.at[pl.ds(p * SEG, SEG)], idx_v)
        _offset_loop(idx_v, SEG, o * _N)
        plsc.subcore_barrier()
        pltpu.async_copy(table_sh.at[idx_v], vals_v, sem).wait()
        pltpu.sync_copy(vals_v, out_ref.at[pl.ds(o * _E + p * SEG, SEG)])

    return k(tableT_flat, idx)


def _sc_gather_pairsT(tableT_flat, candT_flat):
    SEG = _P // 4

    @functools.partial(
        pl.kernel,
        mesh=_mesh(),
        out_type=jax.ShapeDtypeStruct((2 * _H * _P,), jnp.float32),
        scratch_types=[
            pltpu.VMEM((SEG,), jnp.int32),
            pltpu.VMEM((SEG,), jnp.float32),
            pltpu.VMEM_SHARED((_H * _N,), jnp.float32),
            pltpu.SemaphoreType.DMA,
        ],
    )
    def k(table_ref, cand_ref, out_ref, idx_v, vals_v, table_sh, sem):
        wid = lax.axis_index("s") * _NC + lax.axis_index("c")
        sid = lax.axis_index("s")
        f = wid // 4
        part = wid % 4
        s = f // _H
        o = f % _H

        @pl.when(sid == 0)
        def _():
            pltpu.sync_copy(table_ref, table_sh)

        pltpu.sync_copy(cand_ref.at[pl.ds(s * _P + part * SEG, SEG)], idx_v)
        _offset_loop(idx_v, SEG, o * _N)
        plsc.subcore_barrier()
        pltpu.async_copy(table_sh.at[idx_v], vals_v, sem).wait()
        pltpu.sync_copy(vals_v, out_ref.at[pl.ds(f * _P + part * SEG, SEG)])

    return k(tableT_flat, candT_flat)


def _tc_edge1(eaT, x_src, W1a, W1bp, sel):
    TE = 1280
    grid = _E // TE
    dn_t = (((0,), (0,)), ((), ()))

    def body(ea_ref, xs_ref, wa_ref, wb_ref, sel_ref, out_ref):
        h = jnp.maximum(
            lax.dot_general(ea_ref[...], wa_ref[...], dn_t,
                            preferred_element_type=jnp.float32), 0.0)
        w = jnp.dot(h, wb_ref[...], preferred_element_type=jnp.float32)
        xs = xs_ref[...]
        xs4 = jnp.concatenate([xs, xs, xs, xs], axis=1)
        prod = xs4 * w
        out_ref[...] = lax.dot_general(
            sel_ref[...], prod, (((0,), (1,)), ((), ())),
            preferred_element_type=jnp.float32)

    return pl.pallas_call(
        body,
        grid=(grid,),
        in_specs=[
            pl.BlockSpec((_H, TE), lambda i: (0, i)),
            pl.BlockSpec((TE, _IN), lambda i: (i, 0)),
            pl.BlockSpec((4, 512), lambda i: (0, 0)),
            pl.BlockSpec((512, 512), lambda i: (0, 0)),
            pl.BlockSpec((512, _H), lambda i: (0, 0)),
        ],
        out_specs=pl.BlockSpec((_H, TE), lambda i: (0, i)),
        out_shape=jax.ShapeDtypeStruct((_H, _E), jnp.float32),
    )(eaT, x_src, W1a, W1bp, sel)


def _tc_edge2(eaT, h1sT, W2aT, b2aT, W2bpT, b2bpT):
    TE = 16000
    grid = _E // TE

    def body(ea_ref, hs_ref, wa_ref, ba_ref, wb_ref, bb_ref, out_ref):
        g = jnp.maximum(
            jnp.dot(wa_ref[...], ea_ref[...],
                    preferred_element_type=jnp.float32) + ba_ref[...], 0.0)
        w = jnp.dot(wb_ref[...], g,
                    preferred_element_type=jnp.float32) + bb_ref[...]
        hs = hs_ref[...]
        rows = []
        for o in range(_H):
            acc = hs[0:1, :] * w[o * _H:o * _H + 1, :]
            for i in range(1, _H):
                acc = acc + hs[i:i + 1, :] * w[o * _H + i:o * _H + i + 1, :]
            rows.append(acc)
        out_ref[...] = jnp.concatenate(rows, axis=0)

    return pl.pallas_call(
        body,
        grid=(grid,),
        in_specs=[
            pl.BlockSpec((_H, TE), lambda i: (0, i)),
            pl.BlockSpec((_H, TE), lambda i: (0, i)),
            pl.BlockSpec((16, 4), lambda i: (0, 0)),
            pl.BlockSpec((16, 1), lambda i: (0, 0)),
            pl.BlockSpec((16, 16), lambda i: (0, 0)),
            pl.BlockSpec((16, 1), lambda i: (0, 0)),
        ],
        out_specs=pl.BlockSpec((_H, TE), lambda i: (0, i)),
        out_shape=jax.ShapeDtypeStruct((_H, _E), jnp.float32),
    )(eaT, h1sT, W2aT, b2aT, W2bpT, b2bpT)


def _tc_node(partials, xinT, rootT, biasc):
    TN = _N
    grid = 1
    K = xinT.shape[0]

    def body(p0_ref, p1_ref, x_ref, r_ref, b_ref, out_ref):
        acc = (p0_ref[0] + p1_ref[0]
               + jnp.dot(r_ref[...], x_ref[...],
                         preferred_element_type=jnp.float32)
               + b_ref[...])
        out_ref[...] = jnp.maximum(acc, 0.0)

    return pl.pallas_call(
        body,
        grid=(grid,),
        in_specs=[
            pl.BlockSpec((1, _H, TN), lambda i: (0, 0, i)),
            pl.BlockSpec((1, _H, TN), lambda i: (1, 0, i)),
            pl.BlockSpec((K, TN), lambda i: (0, i)),
            pl.BlockSpec((_H, K), lambda i: (0, 0)),
            pl.BlockSpec((_H, 1), lambda i: (0, 0)),
        ],
        out_specs=pl.BlockSpec((_H, TN), lambda i: (0, i)),
        out_shape=jax.ShapeDtypeStruct((_H, _N), jnp.float32),
    )(partials, partials, xinT, rootT, biasc)


def _tc_cand(nfT, We1T, be1c, We2T, be2c):
    TP = _P
    grid = 1

    def body(nf_ref, w1_ref, b1_ref, w2_ref, b2_ref, out_ref):
        hid = jnp.maximum(
            jnp.dot(w1_ref[...], nf_ref[...],
                    preferred_element_type=jnp.float32) + b1_ref[...], 0.0)
        out_ref[...] = jnp.dot(w2_ref[...], hid,
                               preferred_element_type=jnp.float32) + b2_ref[...]

    return pl.pallas_call(
        body,
        grid=(grid,),
        in_specs=[
            pl.BlockSpec((2 * _H, TP), lambda i: (0, i)),
            pl.BlockSpec((_H, 2 * _H), lambda i: (0, 0)),
            pl.BlockSpec((_H, 1), lambda i: (0, 0)),
            pl.BlockSpec((_H, _H), lambda i: (0, 0)),
            pl.BlockSpec((_H, 1), lambda i: (0, 0)),
        ],
        out_specs=pl.BlockSpec((_H, TP), lambda i: (0, i)),
        out_shape=jax.ShapeDtypeStruct((_H, _P), jnp.float32),
    )(nfT, We1T, be1c, We2T, be2c)


def kernel(x, edge_index, edge_attr, override_candidates,
           W1a, b1a, W1b, b1b, root1, bias1,
           W2a, b2a, W2b, b2b, root2, bias2,
           We1, be1, We2, be2):
    src = edge_index[0]
    dst = edge_index[1]

    eaT = edge_attr.T
    xT = x.T
    candT_flat = override_candidates.T.reshape(-1)

    W1bp = W1b.reshape(512, _IN, _H).transpose(0, 2, 1).reshape(512, 512)
    sel = (jnp.arange(512, dtype=jnp.int32)[:, None] // _IN
           == jnp.arange(_H, dtype=jnp.int32)[None, :]).astype(jnp.float32)
    b1bp = b1b.reshape(_IN, _H).T.reshape(1, 512)
    W2aT = W2a.T
    b2aT = b2a.reshape(16, 1)
    W2bpT = W2b.reshape(16, _H, _H).transpose(2, 1, 0).reshape(16, 16)
    b2bpT = b2b.reshape(_H, _H).T.reshape(16, 1)

    zeros_flat = jnp.zeros((_H * _N,), jnp.float32)

    x_src = _sc_gather_rows(x, src, _IN, chunk=1000, n_chunks=5)
    msg1T = _tc_edge1(eaT, x_src, W1a, W1bp, sel)
    part1 = _sc_scatter_add(msg1T.reshape(-1), dst, zeros_flat)
    h1T = _tc_node(part1.reshape(_NC, _H, _N), xT, root1.T,
                   bias1.reshape(_H, 1))

    h1sT = _sc_gather_nodeT(h1T.reshape(-1), src)
    msg2T = _tc_edge2(eaT, h1sT.reshape(_H, _E), W2aT, b2aT, W2bpT, b2bpT)
    part2 = _sc_scatter_add(msg2T.reshape(-1), dst, zeros_flat)
    h2T = _tc_node(part2.reshape(_NC, _H, _N), h1T, root2.T,
                   bias2.reshape(_H, 1))

    nfT = _sc_gather_pairsT(h2T.reshape(-1), candT_flat)
    logitsT = _tc_cand(nfT.reshape(2 * _H, _P), We1.T,
                       be1.reshape(_H, 1), We2.T, be2.reshape(_H, 1))

    return (logitsT.T, override_candidates, h2T.T)
